# Initial kernel scaffold; baseline (speedup 1.0000x reference)
import jax
import jax.numpy as jnp
from jax.experimental import pallas as pl


def kernel(x, edge_index, W_l1, b_l1, W_r1, g1, be1, W_l2, b_l2, W_r2, g2, be2, W_l3, b_l3, W_r3, g3, be3, Wp, bp):
    raise NotImplementedError("write your pallas kernel here")


# speedup vs baseline: 5.0085x; 5.0085x over previous
import functools

import jax
import jax.numpy as jnp
from jax import lax
from jax.experimental import pallas as pl
from jax.experimental.pallas import tpu as pltpu
from jax.experimental.pallas import tpu_sc as plsc

N = 10000
E = 320000

NC = 2
NS = 16
NW = NC * NS
EPW = E // NW
B = 80
CH = EPW // B
RPS_A = 624
RPS_B = N - (NS - 1) * RPS_A
CW = 16


def _for_rows(s, fn):
    @pl.when(s < NS - 1)
    def _():
        fn(pl.multiple_of(s * RPS_A, 8), RPS_A)

    @pl.when(s == NS - 1)
    def _():
        fn(N - RPS_B, RPS_B)


def _agg_body(D, with_cnt, *refs):
    if with_cnt:
        (y, src, dst, zeros, zc, ones, aggp, cntp,
         sem, src_v, dst_v, rows_v, acc, ones_v, cacc) = refs
    else:
        (y, src, dst, zeros, aggp,
         sem, src_v, dst_v, rows_v, acc) = refs
        zc = ones = cntp = ones_v = cacc = None

    c = lax.axis_index("c")
    s = lax.axis_index("s")
    wid = s * NC + c

    def _zero(off, nr):
        pltpu.sync_copy(zeros.at[pl.ds(off, nr)], acc.at[pl.ds(off, nr)])
        if with_cnt:
            pltpu.sync_copy(zc.at[pl.ds(off, nr)], cacc.at[pl.ds(off, nr)])

    _for_rows(s, _zero)
    if with_cnt:
        pltpu.sync_copy(ones, ones_v)
    plsc.subcore_barrier()

    def chunk(i, carry):
        pltpu.sync_copy(src.at[wid, i], src_v)
        pltpu.sync_copy(dst.at[wid, i], dst_v)
        pltpu.async_copy(y.at[src_v], rows_v, sem).wait()
        pltpu.sync_copy(rows_v, acc.at[dst_v], add=True)
        if with_cnt:
            pltpu.sync_copy(ones_v, cacc.at[dst_v], add=True)
        return carry

    lax.fori_loop(0, CH, chunk, 0)
    plsc.subcore_barrier()

    def _export(off, nr):
        pltpu.sync_copy(acc.at[pl.ds(off, nr)], aggp.at[c, pl.ds(off, nr)])
        if with_cnt:
            pltpu.sync_copy(cacc.at[pl.ds(off, nr)], cntp.at[c, pl.ds(off, nr)])

    _for_rows(s, _export)


def _make_agg(D, with_cnt):
    mesh = plsc.VectorSubcoreMesh(
        core_axis_name="c", subcore_axis_name="s",
        num_cores=NC, num_subcores=NS)
    out_type = [jax.ShapeDtypeStruct((NC, N, D), jnp.float32)]
    scratch = [
        pltpu.SemaphoreType.DMA,
        pltpu.VMEM((B,), jnp.int32),
        pltpu.VMEM((B,), jnp.int32),
        pltpu.VMEM((B, D), jnp.float32),
        pltpu.VMEM_SHARED((N, D), jnp.float32),
    ]
    if with_cnt:
        out_type.append(jax.ShapeDtypeStruct((NC, N, CW), jnp.float32))
        scratch += [
            pltpu.VMEM((B, CW), jnp.float32),
            pltpu.VMEM_SHARED((N, CW), jnp.float32),
        ]
    return pl.kernel(
        functools.partial(_agg_body, D, with_cnt),
        out_type=tuple(out_type) if with_cnt else out_type[0],
        mesh=mesh,
        scratch_types=scratch,
        compiler_params=pltpu.CompilerParams(use_tc_tiling_on_sc=False),
        name=f"sc_agg_d{D}" + ("_cnt" if with_cnt else ""),
    )


_agg64_cnt = _make_agg(64, True)
_agg64 = _make_agg(64, False)
_agg128 = _make_agg(128, False)


def _mm_body(x_ref, w_ref, o_ref):
    o_ref[...] = jnp.dot(x_ref[...], w_ref[...],
                         preferred_element_type=jnp.float32)


def _bn_relu(t, g, be):
    mu = jnp.mean(t, axis=0, keepdims=True)
    var = jnp.mean((t - mu) * (t - mu), axis=0, keepdims=True)
    return jnp.maximum(g * (t - mu) * lax.rsqrt(var + 1e-5) + be, 0.0)


def _mean_from_partials(aggp_ref, cntp_ref):
    cnt = cntp_ref[0, :, 0:1] + cntp_ref[1, :, 0:1]
    s = aggp_ref[0, :, :] + aggp_ref[1, :, :]
    return s / jnp.maximum(cnt, 1.0)


def _l1_body(aggp, cntp, xr, b, g, be, o):
    t = _mean_from_partials(aggp, cntp) + b[...] + xr[...]
    o[...] = _bn_relu(t, g[...], be[...])


def _l2_body(aggp, cntp, h_prev, wl, b, wr, g, be, o):
    mean = _mean_from_partials(aggp, cntp)
    t = (jnp.dot(mean, wl[...], preferred_element_type=jnp.float32) + b[...]
         + jnp.dot(h_prev[...], wr[...], preferred_element_type=jnp.float32))
    o[...] = _bn_relu(t, g[...], be[...])


def _l3_body(aggp, cntp, h_prev, wl, b, wr, g, be, wp, bp, o):
    mean = _mean_from_partials(aggp, cntp)
    t = (jnp.dot(mean, wl[...], preferred_element_type=jnp.float32) + b[...]
         + jnp.dot(h_prev[...], wr[...], preferred_element_type=jnp.float32))
    h = _bn_relu(t, g[...], be[...])
    o[...] = jnp.dot(h, wp[...], preferred_element_type=jnp.float32) + bp[...]


def _tc_call(body, out_shape, n_in, name):
    return pl.pallas_call(
        body,
        out_shape=jax.ShapeDtypeStruct(out_shape, jnp.float32),
        name=name,
    )


def kernel(x, edge_index, W_l1, b_l1, W_r1, g1, be1, W_l2, b_l2, W_r2, g2,
           be2, W_l3, b_l3, W_r3, g3, be3, Wp, bp):
    ei = edge_index.astype(jnp.int32)
    src = ei[0].reshape(NW, CH, B)
    dst = ei[1].reshape(NW, CH, B)

    zeros64 = jnp.zeros((N, 64), jnp.float32)
    zeros128 = jnp.zeros((N, 128), jnp.float32)
    zc = jnp.zeros((N, CW), jnp.float32)
    ones = jnp.ones((B, CW), jnp.float32)

    wcat = jnp.concatenate([W_l1.T, W_r1.T], axis=1)
    y = _tc_call(_mm_body, (N, 128), 2, "tc_pre")(x, wcat)
    y1 = y[:, :64]
    xr1 = y[:, 64:]

    aggp1, cntp = _agg64_cnt(y1, src, dst, zeros64, zc, ones)
    h1 = _tc_call(_l1_body, (N, 64), 6, "tc_l1")(
        aggp1, cntp, xr1, b_l1[None], g1[None], be1[None])

    aggp2 = _agg64(h1, src, dst, zeros64)
    h2 = _tc_call(_l2_body, (N, 128), 8, "tc_l2")(
        aggp2, cntp, h1, W_l2.T, b_l2[None], W_r2.T, g2[None], be2[None])

    aggp3 = _agg128(h2, src, dst, zeros128)
    out = _tc_call(_l3_body, (N, 2), 10, "tc_l3")(
        aggp3, cntp, h2, W_l3.T, b_l3[None], W_r3.T, g3[None], be3[None],
        Wp.T, bp[None])
    return out


# speedup vs baseline: 8.9825x; 1.7935x over previous
import functools

import jax
import jax.numpy as jnp
from jax import lax
from jax.experimental import pallas as pl
from jax.experimental.pallas import tpu as pltpu
from jax.experimental.pallas import tpu_sc as plsc

N = 10000
E = 320000

NC = 2
NS = 16
NW = NC * NS
EPW = E // NW
B = 80
CH = EPW // B
RPS_A = 624
RPS_B = N - (NS - 1) * RPS_A
CW = 16


def _for_rows(s, fn):
    @pl.when(s < NS - 1)
    def _():
        fn(pl.multiple_of(s * RPS_A, 8), RPS_A)

    @pl.when(s == NS - 1)
    def _():
        fn(N - RPS_B, RPS_B)


def _agg_body(D, with_cnt, *refs):
    if with_cnt:
        (y, src, dst, zeros, zc, ones, aggp, cntp,
         gsem, isem, s0, s1, d0, d1, r0, r1, acc, ones_v, cacc) = refs
    else:
        (y, src, dst, zeros, aggp,
         gsem, isem, s0, s1, d0, d1, r0, r1, acc) = refs
        zc = ones = cntp = ones_v = cacc = None

    c = lax.axis_index("c")
    s = lax.axis_index("s")
    wid = s * NC + c

    def _zero(off, nr):
        pltpu.sync_copy(zeros.at[pl.ds(off, nr)], acc.at[pl.ds(off, nr)])
        if with_cnt:
            pltpu.sync_copy(zc.at[pl.ds(off, nr)], cacc.at[pl.ds(off, nr)])

    _for_rows(s, _zero)
    if with_cnt:
        pltpu.sync_copy(ones, ones_v)
    plsc.subcore_barrier()

    pltpu.sync_copy(src.at[wid, 0], s0)
    pltpu.sync_copy(dst.at[wid, 0], d0)
    pltpu.async_copy(src.at[wid, 1], s1, isem)
    pltpu.async_copy(dst.at[wid, 1], d1, isem)
    pltpu.async_copy(y.at[s0], r0, gsem)

    def step(i, scur, dcur, rcur, snxt, dnxt, rnxt):
        pltpu.make_async_copy(y.at[scur], rcur, gsem).wait()
        pltpu.make_async_copy(src.at[wid, i + 1], snxt, isem).wait()
        pltpu.make_async_copy(dst.at[wid, i + 1], dnxt, isem).wait()
        pltpu.async_copy(y.at[snxt], rnxt, gsem)
        if with_cnt:
            pltpu.sync_copy(ones_v, cacc.at[dcur], add=True)
        pltpu.sync_copy(rcur, acc.at[dcur], add=True)
        pltpu.async_copy(src.at[wid, i + 2], scur, isem)
        pltpu.async_copy(dst.at[wid, i + 2], dcur, isem)

    def pair(k, carry):
        i0 = 2 * k
        step(i0, s0, d0, r0, s1, d1, r1)
        step(i0 + 1, s1, d1, r1, s0, d0, r0)
        return carry

    lax.fori_loop(0, CH // 2, pair, 0)

    pltpu.make_async_copy(y.at[s0], r0, gsem).wait()
    if with_cnt:
        pltpu.sync_copy(ones_v, cacc.at[d0], add=True)
    pltpu.sync_copy(r0, acc.at[d0], add=True)
    pltpu.make_async_copy(src.at[wid, CH], s1, isem).wait()
    pltpu.make_async_copy(dst.at[wid, CH], d1, isem).wait()
    plsc.subcore_barrier()

    def _export(off, nr):
        pltpu.sync_copy(acc.at[pl.ds(off, nr)], aggp.at[c, pl.ds(off, nr)])
        if with_cnt:
            pltpu.sync_copy(cacc.at[pl.ds(off, nr)], cntp.at[c, pl.ds(off, nr)])

    _for_rows(s, _export)


def _make_agg(D, with_cnt):
    mesh = plsc.VectorSubcoreMesh(
        core_axis_name="c", subcore_axis_name="s",
        num_cores=NC, num_subcores=NS)
    out_type = [jax.ShapeDtypeStruct((NC, N, D), jnp.float32)]
    scratch = [
        pltpu.SemaphoreType.DMA,
        pltpu.SemaphoreType.DMA,
        pltpu.VMEM((B,), jnp.int32),
        pltpu.VMEM((B,), jnp.int32),
        pltpu.VMEM((B,), jnp.int32),
        pltpu.VMEM((B,), jnp.int32),
        pltpu.VMEM((B, D), jnp.float32),
        pltpu.VMEM((B, D), jnp.float32),
        pltpu.VMEM_SHARED((N, D), jnp.float32),
    ]
    if with_cnt:
        out_type.append(jax.ShapeDtypeStruct((NC, N, CW), jnp.float32))
        scratch += [
            pltpu.VMEM((B, CW), jnp.float32),
            pltpu.VMEM_SHARED((N, CW), jnp.float32),
        ]
    return pl.kernel(
        functools.partial(_agg_body, D, with_cnt),
        out_type=tuple(out_type) if with_cnt else out_type[0],
        mesh=mesh,
        scratch_types=scratch,
        compiler_params=pltpu.CompilerParams(use_tc_tiling_on_sc=False),
        name=f"sc_agg_d{D}" + ("_cnt" if with_cnt else ""),
    )


_agg64_cnt = _make_agg(64, True)
_agg64 = _make_agg(64, False)
_agg128 = _make_agg(128, False)


def _mm_body(x_ref, w_ref, o_ref):
    o_ref[...] = jnp.dot(x_ref[...], w_ref[...],
                         preferred_element_type=jnp.float32)


def _bn_relu(t, g, be):
    mu = jnp.mean(t, axis=0, keepdims=True)
    var = jnp.mean((t - mu) * (t - mu), axis=0, keepdims=True)
    return jnp.maximum(g * (t - mu) * lax.rsqrt(var + 1e-5) + be, 0.0)


def _mean_from_partials(aggp_ref, cntp_ref):
    cnt = cntp_ref[0, :, 0:1] + cntp_ref[1, :, 0:1]
    s = aggp_ref[0, :, :] + aggp_ref[1, :, :]
    return s / jnp.maximum(cnt, 1.0)


def _l1_body(aggp, cntp, xr, b, g, be, o):
    t = _mean_from_partials(aggp, cntp) + b[...] + xr[...]
    o[...] = _bn_relu(t, g[...], be[...])


def _l2_body(aggp, cntp, h_prev, wl, b, wr, g, be, o):
    mean = _mean_from_partials(aggp, cntp)
    t = (jnp.dot(mean, wl[...], preferred_element_type=jnp.float32) + b[...]
         + jnp.dot(h_prev[...], wr[...], preferred_element_type=jnp.float32))
    o[...] = _bn_relu(t, g[...], be[...])


def _l3_body(aggp, cntp, h_prev, wl, b, wr, g, be, wp, bp, o):
    mean = _mean_from_partials(aggp, cntp)
    t = (jnp.dot(mean, wl[...], preferred_element_type=jnp.float32) + b[...]
         + jnp.dot(h_prev[...], wr[...], preferred_element_type=jnp.float32))
    h = _bn_relu(t, g[...], be[...])
    o[...] = jnp.dot(h, wp[...], preferred_element_type=jnp.float32) + bp[...]


def _tc_call(body, out_shape, n_in, name):
    return pl.pallas_call(
        body,
        out_shape=jax.ShapeDtypeStruct(out_shape, jnp.float32),
        name=name,
    )


def kernel(x, edge_index, W_l1, b_l1, W_r1, g1, be1, W_l2, b_l2, W_r2, g2,
           be2, W_l3, b_l3, W_r3, g3, be3, Wp, bp):
    ei = edge_index.astype(jnp.int32)
    pad = jnp.zeros((NW, 2, B), jnp.int32)
    src = jnp.concatenate([ei[0].reshape(NW, CH, B), pad], axis=1)
    dst = jnp.concatenate([ei[1].reshape(NW, CH, B), pad], axis=1)

    zeros64 = jnp.zeros((N, 64), jnp.float32)
    zeros128 = jnp.zeros((N, 128), jnp.float32)
    zc = jnp.zeros((N, CW), jnp.float32)
    ones = jnp.ones((B, CW), jnp.float32)

    wcat = jnp.concatenate([W_l1.T, W_r1.T], axis=1)
    y = _tc_call(_mm_body, (N, 128), 2, "tc_pre")(x, wcat)
    y1 = y[:, :64]
    xr1 = y[:, 64:]

    aggp1, cntp = _agg64_cnt(y1, src, dst, zeros64, zc, ones)
    h1 = _tc_call(_l1_body, (N, 64), 6, "tc_l1")(
        aggp1, cntp, xr1, b_l1[None], g1[None], be1[None])

    aggp2 = _agg64(h1, src, dst, zeros64)
    h2 = _tc_call(_l2_body, (N, 128), 8, "tc_l2")(
        aggp2, cntp, h1, W_l2.T, b_l2[None], W_r2.T, g2[None], be2[None])

    aggp3 = _agg128(h2, src, dst, zeros128)
    out = _tc_call(_l3_body, (N, 2), 10, "tc_l3")(
        aggp3, cntp, h2, W_l3.T, b_l3[None], W_r3.T, g3[None], be3[None],
        Wp.T, bp[None])
    return out


# speedup vs baseline: 9.0383x; 1.0062x over previous
import functools

import jax
import jax.numpy as jnp
from jax import lax
from jax.experimental import pallas as pl
from jax.experimental.pallas import tpu as pltpu
from jax.experimental.pallas import tpu_sc as plsc

N = 10000
E = 320000

NC = 2
NS = 16
NW = NC * NS
EPW = E // NW
B = 80
CH = EPW // B
RPS_A = 624
RPS_B = N - (NS - 1) * RPS_A
CW = 16


def _for_rows(s, fn):
    @pl.when(s < NS - 1)
    def _():
        fn(pl.multiple_of(s * RPS_A, 8), RPS_A)

    @pl.when(s == NS - 1)
    def _():
        fn(N - RPS_B, RPS_B)


def _agg_body(D, with_cnt, *refs):
    if with_cnt:
        (y, src, dst, zeros, zc, ones, aggp, cntp,
         gsem, isem, ssem, s0, s1, s2, s3, d0, d1, d2, d3,
         r0, r1, r2, r3, acc, ones_v, cacc) = refs
    else:
        (y, src, dst, zeros, aggp,
         gsem, isem, ssem, s0, s1, s2, s3, d0, d1, d2, d3,
         r0, r1, r2, r3, acc) = refs
        zc = ones = cntp = ones_v = cacc = None
    sb = (s0, s1, s2, s3)
    db = (d0, d1, d2, d3)
    rb = (r0, r1, r2, r3)

    c = lax.axis_index("c")
    s = lax.axis_index("s")
    wid = s * NC + c

    def _zero(off, nr):
        pltpu.sync_copy(zeros.at[pl.ds(off, nr)], acc.at[pl.ds(off, nr)])
        if with_cnt:
            pltpu.sync_copy(zc.at[pl.ds(off, nr)], cacc.at[pl.ds(off, nr)])

    _for_rows(s, _zero)
    if with_cnt:
        pltpu.sync_copy(ones, ones_v)
    plsc.subcore_barrier()

    pltpu.sync_copy(src.at[wid, 0], s0)
    pltpu.sync_copy(dst.at[wid, 0], d0)
    pltpu.async_copy(src.at[wid, 1], s1, isem)
    pltpu.async_copy(dst.at[wid, 1], d1, isem)
    pltpu.async_copy(y.at[s0], r0, gsem)

    def step(i, k, p):
        scur, dcur, rcur = sb[p], db[p], rb[p]
        snxt, dnxt, rnxt = sb[(p + 1) % 4], db[(p + 1) % 4], rb[(p + 1) % 4]
        sprv, dprv, rprv = sb[(p - 1) % 4], db[(p - 1) % 4], rb[(p - 1) % 4]
        s2n, d2n = sb[(p + 2) % 4], db[(p + 2) % 4]
        pltpu.make_async_copy(y.at[scur], rcur, gsem).wait()
        pltpu.make_async_copy(src.at[wid, i + 1], snxt, isem).wait()
        pltpu.make_async_copy(dst.at[wid, i + 1], dnxt, isem).wait()
        pltpu.async_copy(y.at[snxt], rnxt, gsem)

        def wait_prev_scatter():
            pltpu.make_async_copy(rprv, acc.at[dprv], ssem).wait()
            if with_cnt:
                pltpu.make_async_copy(ones_v, cacc.at[dprv], ssem).wait()

        if p == 0:
            @pl.when(k > 0)
            def _():
                wait_prev_scatter()
        else:
            wait_prev_scatter()
        pltpu.async_copy(rcur, acc.at[dcur], ssem, add=True)
        if with_cnt:
            pltpu.async_copy(ones_v, cacc.at[dcur], ssem, add=True)
        pltpu.async_copy(src.at[wid, i + 2], s2n, isem)
        pltpu.async_copy(dst.at[wid, i + 2], d2n, isem)

    def quad(k, carry):
        i0 = 4 * k
        step(i0, k, 0)
        step(i0 + 1, k, 1)
        step(i0 + 2, k, 2)
        step(i0 + 3, k, 3)
        return carry

    lax.fori_loop(0, (CH - 1) // 4, quad, 0)

    pltpu.make_async_copy(y.at[sb[0]], rb[0], gsem).wait()
    pltpu.make_async_copy(rb[3], acc.at[db[3]], ssem).wait()
    if with_cnt:
        pltpu.make_async_copy(ones_v, cacc.at[db[3]], ssem).wait()
    if with_cnt:
        pltpu.sync_copy(ones_v, cacc.at[db[0]], add=True)
    pltpu.sync_copy(rb[0], acc.at[db[0]], add=True)
    pltpu.make_async_copy(src.at[wid, CH], sb[1], isem).wait()
    pltpu.make_async_copy(dst.at[wid, CH], db[1], isem).wait()
    plsc.subcore_barrier()

    def _export(off, nr):
        pltpu.sync_copy(acc.at[pl.ds(off, nr)], aggp.at[c, pl.ds(off, nr)])
        if with_cnt:
            pltpu.sync_copy(cacc.at[pl.ds(off, nr)], cntp.at[c, pl.ds(off, nr)])

    _for_rows(s, _export)


def _make_agg(D, with_cnt):
    mesh = plsc.VectorSubcoreMesh(
        core_axis_name="c", subcore_axis_name="s",
        num_cores=NC, num_subcores=NS)
    out_type = [jax.ShapeDtypeStruct((NC, N, D), jnp.float32)]
    scratch = (
        [pltpu.SemaphoreType.DMA] * 3
        + [pltpu.VMEM((B,), jnp.int32)] * 8
        + [pltpu.VMEM((B, D), jnp.float32)] * 4
        + [pltpu.VMEM_SHARED((N, D), jnp.float32)]
    )
    if with_cnt:
        out_type.append(jax.ShapeDtypeStruct((NC, N, CW), jnp.float32))
        scratch += [
            pltpu.VMEM((B, CW), jnp.float32),
            pltpu.VMEM_SHARED((N, CW), jnp.float32),
        ]
    return pl.kernel(
        functools.partial(_agg_body, D, with_cnt),
        out_type=tuple(out_type) if with_cnt else out_type[0],
        mesh=mesh,
        scratch_types=scratch,
        compiler_params=pltpu.CompilerParams(use_tc_tiling_on_sc=False),
        name=f"sc_agg_d{D}" + ("_cnt" if with_cnt else ""),
    )


_agg64_cnt = _make_agg(64, True)
_agg64 = _make_agg(64, False)
_agg128 = _make_agg(128, False)


def _mm_body(x_ref, w_ref, o_ref):
    o_ref[...] = jnp.dot(x_ref[...], w_ref[...],
                         preferred_element_type=jnp.float32)


def _bn_relu(t, g, be):
    mu = jnp.mean(t, axis=0, keepdims=True)
    var = jnp.mean((t - mu) * (t - mu), axis=0, keepdims=True)
    return jnp.maximum(g * (t - mu) * lax.rsqrt(var + 1e-5) + be, 0.0)


def _mean_from_partials(aggp_ref, cntp_ref):
    cnt = cntp_ref[0, :, 0:1] + cntp_ref[1, :, 0:1]
    s = aggp_ref[0, :, :] + aggp_ref[1, :, :]
    return s / jnp.maximum(cnt, 1.0)


def _l1_body(aggp, cntp, xr, b, g, be, o):
    t = _mean_from_partials(aggp, cntp) + b[...] + xr[...]
    o[...] = _bn_relu(t, g[...], be[...])


def _l2_body(aggp, cntp, h_prev, wl, b, wr, g, be, o):
    mean = _mean_from_partials(aggp, cntp)
    t = (jnp.dot(mean, wl[...], preferred_element_type=jnp.float32) + b[...]
         + jnp.dot(h_prev[...], wr[...], preferred_element_type=jnp.float32))
    o[...] = _bn_relu(t, g[...], be[...])


def _l3_body(aggp, cntp, h_prev, wl, b, wr, g, be, wp, bp, o):
    mean = _mean_from_partials(aggp, cntp)
    t = (jnp.dot(mean, wl[...], preferred_element_type=jnp.float32) + b[...]
         + jnp.dot(h_prev[...], wr[...], preferred_element_type=jnp.float32))
    h = _bn_relu(t, g[...], be[...])
    o[...] = jnp.dot(h, wp[...], preferred_element_type=jnp.float32) + bp[...]


def _tc_call(body, out_shape, n_in, name):
    return pl.pallas_call(
        body,
        out_shape=jax.ShapeDtypeStruct(out_shape, jnp.float32),
        name=name,
    )


def kernel(x, edge_index, W_l1, b_l1, W_r1, g1, be1, W_l2, b_l2, W_r2, g2,
           be2, W_l3, b_l3, W_r3, g3, be3, Wp, bp):
    ei = edge_index.astype(jnp.int32)
    pad = jnp.zeros((NW, 2, B), jnp.int32)
    src = jnp.concatenate([ei[0].reshape(NW, CH, B), pad], axis=1)
    dst = jnp.concatenate([ei[1].reshape(NW, CH, B), pad], axis=1)

    zeros64 = jnp.zeros((N, 64), jnp.float32)
    zeros128 = jnp.zeros((N, 128), jnp.float32)
    zc = jnp.zeros((N, CW), jnp.float32)
    ones = jnp.ones((B, CW), jnp.float32)

    wcat = jnp.concatenate([W_l1.T, W_r1.T], axis=1)
    y = _tc_call(_mm_body, (N, 128), 2, "tc_pre")(x, wcat)
    y1 = y[:, :64]
    xr1 = y[:, 64:]

    aggp1, cntp = _agg64_cnt(y1, src, dst, zeros64, zc, ones)
    h1 = _tc_call(_l1_body, (N, 64), 6, "tc_l1")(
        aggp1, cntp, xr1, b_l1[None], g1[None], be1[None])

    aggp2 = _agg64(h1, src, dst, zeros64)
    h2 = _tc_call(_l2_body, (N, 128), 8, "tc_l2")(
        aggp2, cntp, h1, W_l2.T, b_l2[None], W_r2.T, g2[None], be2[None])

    aggp3 = _agg128(h2, src, dst, zeros128)
    out = _tc_call(_l3_body, (N, 2), 10, "tc_l3")(
        aggp3, cntp, h2, W_l3.T, b_l3[None], W_r3.T, g3[None], be3[None],
        Wp.T, bp[None])
    return out
